# Initial kernel scaffold; baseline (speedup 1.0000x reference)
import jax
import jax.numpy as jnp
from jax.experimental import pallas as pl


def kernel(words, heads_ids, sent_id, emb_table, W1, b1, W2, b2, W3, b3, Cw1, Cb1, Cw2, Cb2):
    raise NotImplementedError("write your pallas kernel here")


# speedup vs baseline: 106.4090x; 106.4090x over previous
import jax
import jax.numpy as jnp
from jax.experimental import pallas as pl
from jax.experimental.pallas import tpu as pltpu

_NEG_INF = float("-inf")
_CHUNK = 128
_PAD = 8


def _matmul_kernel(x_ref, w_ref, b_ref, o_ref):
    o_ref[...] = (
        jnp.dot(x_ref[...], w_ref[...], preferred_element_type=jnp.float32)
        + b_ref[...]
    )


def _project(x, w, b, bm):
    m, k = x.shape
    bm = min(bm, m)
    n = w.shape[1]
    return pl.pallas_call(
        _matmul_kernel,
        grid=(m // bm,),
        in_specs=[
            pl.BlockSpec((bm, k), lambda i: (i, 0)),
            pl.BlockSpec((k, n), lambda i: (0, 0)),
            pl.BlockSpec((1, n), lambda i: (0, 0)),
        ],
        out_specs=pl.BlockSpec((bm, n), lambda i: (i, 0)),
        out_shape=jax.ShapeDtypeStruct((m, n), jnp.float32),
    )(x, w, b)


def _span_kernel(
    starts_ref, lengths_ref, hids_ref, maxlen_ref,
    wp_ref, hp_ref, ep_ref, w2_ref, b2_ref, w3_ref, b3_ref,
    k0_ref, k1_ref, k2_ref, cb1_ref, m0_ref, m1_ref, m2_ref, cb2_ref,
    out_ref, h3buf,
):
    h = pl.program_id(0)
    start = starts_ref[h]
    length = lengths_ref[h]
    hid = hids_ref[h]
    maxlen = maxlen_ref[0]
    end = start + length
    b0 = start // _CHUNK
    nblk = (end + _CHUNK - 1) // _CHUNK - b0

    h3dim = w3_ref.shape[1]
    n_emb = ep_ref.shape[0]

    out_ref[...] = jnp.full(out_ref.shape, _NEG_INF, jnp.float32)
    h3buf[0:_PAD, :] = jnp.zeros((_PAD, h3dim), jnp.float32)

    hp = hp_ref[0]

    def ffnn_body(i, carry):
        w0 = (b0 + i) * _CHUNK
        wp = wp_ref[pl.ds(w0, _CHUNK), :]
        wvec = w0 + jax.lax.broadcasted_iota(jnp.int32, (_CHUNK, 1), 0)
        eid = hid - wvec + 63
        eid = jnp.where((eid < 0) | (eid > 126), 127, eid)
        onehot = (
            eid == jax.lax.broadcasted_iota(jnp.int32, (_CHUNK, n_emb), 1)
        ).astype(jnp.float32)
        emb = jnp.dot(onehot, ep_ref[...], preferred_element_type=jnp.float32)
        h1 = jnp.maximum(wp + hp + emb, 0.0)
        h2 = jnp.maximum(
            jnp.dot(h1, w2_ref[...], preferred_element_type=jnp.float32)
            + b2_ref[...],
            0.0,
        )
        h3 = (
            jnp.dot(h2, w3_ref[...], preferred_element_type=jnp.float32)
            + b3_ref[...]
        )
        h3 = jnp.where((wvec >= start) & (wvec < end), h3, 0.0)
        h3buf[pl.ds(_PAD + i * _CHUNK, _CHUNK), :] = h3
        return carry

    jax.lax.fori_loop(0, nblk, ffnn_body, 0)
    h3buf[pl.ds(_PAD + nblk * _CHUNK, _PAD), :] = jnp.zeros((_PAD, h3dim), jnp.float32)

    k0 = k0_ref[...]
    k1 = k1_ref[...]
    k2 = k2_ref[...]
    cb1 = cb1_ref[...]
    m0 = m0_ref[...]
    m1 = m1_ref[...]
    m2 = m2_ref[...]
    cb2 = cb2_ref[...]

    def conv_body(i, carry):
        a = h3buf[pl.ds(i * _CHUNK, _CHUNK + 2 * _PAD), :]
        y = (
            jnp.dot(a[0 : _CHUNK + 14], k0, preferred_element_type=jnp.float32)
            + jnp.dot(a[1 : _CHUNK + 15], k1, preferred_element_type=jnp.float32)
            + jnp.dot(a[2 : _CHUNK + 16], k2, preferred_element_type=jnp.float32)
            + cb1
        )
        g = ((b0 + i) * _CHUNK - _PAD + 1 - start) + jax.lax.broadcasted_iota(
            jnp.int32, (_CHUNK + 14, 1), 0
        )
        y = jnp.where((g >= 0) & (g < maxlen), y, 0.0)
        r = (
            jnp.dot(y[6 : 6 + _CHUNK], m0, preferred_element_type=jnp.float32)
            + jnp.dot(y[7 : 7 + _CHUNK], m1, preferred_element_type=jnp.float32)
            + jnp.dot(y[8 : 8 + _CHUNK], m2, preferred_element_type=jnp.float32)
            + cb2
        )
        w0 = (b0 + i) * _CHUNK
        wvec = w0 + jax.lax.broadcasted_iota(jnp.int32, (_CHUNK, 1), 0)
        r = jnp.where((wvec >= start) & (wvec < end), r, _NEG_INF)
        out_ref[0, pl.ds(w0, _CHUNK), :] = r
        return carry

    jax.lax.fori_loop(0, nblk, conv_body, 0)


def kernel(words, heads_ids, sent_id, emb_table, W1, b1, W2, b2, W3, b3,
           Cw1, Cb1, Cw2, Cb2):
    n_words, d = words.shape
    n_heads = heads_ids.shape[0]

    heads_ids = heads_ids.astype(jnp.int32)
    head_sent = sent_id[heads_ids]
    starts = jnp.searchsorted(sent_id, head_sent, side="left").astype(jnp.int32)
    ends = jnp.searchsorted(sent_id, head_sent, side="right").astype(jnp.int32)
    lengths = ends - starts
    maxlen = jnp.max(lengths).reshape(1).astype(jnp.int32)

    w1a_t = W1[:, :d].T
    w1b_t = W1[:, d : 2 * d].T
    w1c_t = W1[:, 2 * d :].T

    zero_bias = jnp.zeros((1, d), jnp.float32)
    wordsproj = _project(words, w1b_t, zero_bias, 128)
    headproj = _project(words[heads_ids], w1a_t, b1[None], 128)
    embproj = _project(emb_table, w1c_t, zero_bias, 128)

    conv_taps = [Cw1[:, :, t].T for t in range(3)] + [Cw2[:, :, t].T for t in range(3)]

    n_emb = emb_table.shape[0]
    d2 = W2.shape[0]
    d3 = W3.shape[0]
    c1 = Cw1.shape[0]
    c2 = Cw2.shape[0]

    def _full(shape):
        return pl.BlockSpec(shape, lambda h, *_: (0,) * len(shape))

    grid_spec = pltpu.PrefetchScalarGridSpec(
        num_scalar_prefetch=4,
        grid=(n_heads,),
        in_specs=[
            _full((n_words, d)),
            pl.BlockSpec((1, 1, d), lambda h, *_: (h, 0, 0)),
            _full((n_emb, d)),
            _full((d, d2)),
            _full((1, d2)),
            _full((d2, d3)),
            _full((1, d3)),
            _full((d3, c1)),
            _full((d3, c1)),
            _full((d3, c1)),
            _full((1, c1)),
            _full((c1, c2)),
            _full((c1, c2)),
            _full((c1, c2)),
            _full((1, c2)),
        ],
        out_specs=pl.BlockSpec((1, n_words, 2), lambda h, *_: (h, 0, 0)),
        scratch_shapes=[pltpu.VMEM((2 * _PAD + n_words, d3), jnp.float32)],
    )
    out = pl.pallas_call(
        _span_kernel,
        grid_spec=grid_spec,
        out_shape=jax.ShapeDtypeStruct((n_heads, n_words, 2), jnp.float32),
    )(
        starts, lengths, heads_ids, maxlen,
        wordsproj, headproj[:, None, :], embproj,
        W2.T, b2[None], W3.T, b3[None],
        conv_taps[0], conv_taps[1], conv_taps[2], Cb1[None],
        conv_taps[3], conv_taps[4], conv_taps[5], Cb2[None],
    )
    return out
